# baseline (speedup 1.0000x reference)
import functools

import jax
import jax.numpy as jnp
from jax import lax
from jax.experimental import pallas as pl
from jax.experimental.pallas import tpu as pltpu
from jax.experimental.pallas import tpu_sc as plsc

DIM = 16
CTX = 50
LANES = 16
NUM_CORES = 2
NUM_SUBCORES = 16
NUM_WORKERS = NUM_CORES * NUM_SUBCORES

TC_W = 4096


def _transpose_table(wt, V):
    grid = -(-V // TC_W)

    def body(in_ref, out_ref):
        x = in_ref[...]
        out_ref[...] = (x.reshape(DIM, TC_W // 8, 8)
                        .transpose(1, 2, 0)
                        .reshape(TC_W // 8, 128))

    return pl.pallas_call(
        body,
        grid=(grid,),
        in_specs=[pl.BlockSpec((DIM, TC_W), lambda i: (0, i))],
        out_specs=pl.BlockSpec((TC_W // 8, 128), lambda i: (i, 0)),
        out_shape=jax.ShapeDtypeStruct((V // 8, 128), jnp.float32),
    )(wt)


def _make_sc_kernel(B, V):
    b_per_w = B // NUM_WORKERS
    n_groups = b_per_w // LANES
    rows_per_group = LANES * CTX
    chunks = [(o, min(128, rows_per_group - o))
              for o in range(0, rows_per_group, 128)]
    n_cg = b_per_w * DIM // 128

    mesh = plsc.VectorSubcoreMesh(core_axis_name="c", subcore_axis_name="s")

    @functools.partial(
        pl.kernel,
        out_type=jax.ShapeDtypeStruct((B * CTX,), jnp.float32),
        mesh=mesh,
        compiler_params=pltpu.CompilerParams(
            needs_layout_passes=False, use_tc_tiling_on_sc=False),
        scratch_types=[
            pltpu.VMEM((b_per_w,), jnp.int32),
            pltpu.VMEM((b_per_w, DIM), jnp.float32),
            pltpu.VMEM((b_per_w * CTX,), jnp.int32),
            pltpu.VMEM((rows_per_group, DIM), jnp.float32),
            pltpu.VMEM((rows_per_group, DIM), jnp.float32),
            pltpu.VMEM((rows_per_group,), jnp.float32),
            pltpu.VMEM((rows_per_group,), jnp.float32),
            pltpu.SemaphoreType.DMA,
            pltpu.SemaphoreType.DMA,
            pltpu.SemaphoreType.DMA,
            pltpu.SemaphoreType.DMA,
            pltpu.SemaphoreType.DMA,
        ],
    )
    def word_embed(center_hbm, context_hbm, wc_hbm, wx_hbm, out_hbm,
                   cidx_v, crows_v, xidx_v, xrows0, xrows1,
                   outv0, outv1, csem, gs0, gs1, os0, os1):
        wid = lax.axis_index("s") * NUM_CORES + lax.axis_index("c")
        base = wid * b_per_w
        xrows = (xrows0, xrows1)
        outvs = (outv0, outv1)
        gsems = (gs0, gs1)
        osems = (os0, os1)
        lane = lax.iota(jnp.int32, LANES)

        pltpu.sync_copy(center_hbm.at[pl.ds(base, b_per_w)], cidx_v)
        pltpu.sync_copy(context_hbm.at[pl.ds(base * CTX, b_per_w * CTX)],
                        xidx_v)

        cdescs = [pltpu.async_copy(wc_hbm.at[cidx_v], crows_v, csem)]

        def fire_group(g, buf):
            descs = []
            for off, n in chunks:
                idx = xidx_v.at[pl.ds(g * rows_per_group + off, n)]
                descs.append(pltpu.async_copy(
                    wx_hbm.at[idx], xrows[buf].at[pl.ds(off, n)],
                    gsems[buf]))
            return descs

        def compute_group(g, buf):
            rows = xrows[buf]
            ov = outvs[buf]
            ccols = [plsc.load_gather(
                crows_v, [g * LANES + lane, jnp.full((LANES,), d, jnp.int32)])
                for d in range(DIM)]

            def body(m, _):
                row_idx = lane * CTX + m
                acc = jnp.zeros((LANES,), jnp.float32)
                for d in range(DIM):
                    xcol = plsc.load_gather(
                        rows, [row_idx, jnp.full((LANES,), d, jnp.int32)])
                    acc = acc + ccols[d] * xcol
                plsc.store_scatter(ov, [row_idx], acc)
                return _

            lax.fori_loop(0, CTX, body, 0, unroll=2)

        inflight = {}
        out_descs = {}
        inflight[0] = fire_group(0, 0)
        for dsc in cdescs:
            dsc.wait()
        for g in range(n_groups):
            buf = g % 2
            if g + 1 < n_groups:
                inflight[g + 1] = fire_group(g + 1, (g + 1) % 2)
            for dsc in inflight.pop(g):
                dsc.wait()
            if g - 2 in out_descs:
                out_descs.pop(g - 2).wait()
            compute_group(g, buf)
            out_descs[g] = pltpu.async_copy(
                outvs[buf],
                out_hbm.at[pl.ds((base + g * LANES) * CTX, rows_per_group)],
                osems[buf])
        for dsc in out_descs.values():
            dsc.wait()

    return word_embed


def kernel(center, context, W_center, W_context):
    B = center.shape[0]
    V = W_center.shape[0]
    wx_rm = _transpose_table(W_context.T, V)
    wc_rm = _transpose_table(W_center.T, V)
    k = _make_sc_kernel(B, V)
    out_flat = k(center.reshape(B).astype(jnp.int32),
                 context.reshape(B * CTX).astype(jnp.int32),
                 wc_rm.reshape(V, DIM),
                 wx_rm.reshape(V, DIM))
    return out_flat.reshape(B, 1, CTX)


# speedup vs baseline: 6.5439x; 6.5439x over previous
import functools

import jax
import jax.numpy as jnp
from jax import lax
from jax.experimental import pallas as pl
from jax.experimental.pallas import tpu as pltpu
from jax.experimental.pallas import tpu_sc as plsc

DIM = 16
CTX = 50
LANES = 16
NUM_CORES = 2
NUM_SUBCORES = 16
NUM_WORKERS = NUM_CORES * NUM_SUBCORES

VB = 128


def _make_relayout(B, V):
    n_full = V // VB
    per_w = n_full // NUM_WORKERS
    extra = n_full % NUM_WORKERS
    tail = V % VB
    b_per_w = B // NUM_WORKERS

    mesh = plsc.VectorSubcoreMesh(core_axis_name="c", subcore_axis_name="s")

    @functools.partial(
        pl.kernel,
        out_type=(jax.ShapeDtypeStruct((V * DIM,), jnp.float32),
                  jax.ShapeDtypeStruct((B * DIM,), jnp.float32)),
        name="relayout",
        mesh=mesh,
        compiler_params=pltpu.CompilerParams(
            needs_layout_passes=False, use_tc_tiling_on_sc=True),
        scratch_types=[
            pltpu.VMEM((DIM, VB), jnp.float32),
            pltpu.VMEM((DIM, VB), jnp.float32),
            pltpu.VMEM((VB * DIM,), jnp.float32),
            pltpu.VMEM((VB * DIM,), jnp.float32),
            pltpu.VMEM((b_per_w,), jnp.int32),
            pltpu.VMEM((DIM, 16 * VB), jnp.float32),
            pltpu.VMEM((b_per_w * DIM,), jnp.float32),
            pltpu.VMEM((V % VB if V % VB else 8, DIM), jnp.float32),
            pltpu.SemaphoreType.DMA,
            pltpu.SemaphoreType.DMA,
            pltpu.SemaphoreType.DMA,
            pltpu.SemaphoreType.DMA,
            pltpu.SemaphoreType.DMA,
        ],
    )
    def relayout(wx_t, wc_t, center_hbm, wtail_hbm, wx_lin, crows_lin,
                 in0, in1, ob0, ob1, cidx_v, cblk_v, crows_v, tailb,
                 is0, is1, os0, os1, csem):
        wid = lax.axis_index("s") * NUM_CORES + lax.axis_index("c")
        lane = lax.iota(jnp.int32, LANES)
        inb = (in0, in1)
        outb = (ob0, ob1)
        isem = (is0, is1)
        osem = (os0, os1)
        lo = wid * per_w

        def transpose_block(src, dst, nv):
            for u in range(nv // LANES):
                for d in range(DIM):
                    vals = src[d, pl.ds(u * LANES, LANES)]
                    plsc.store_scatter(
                        dst, [lane * DIM + (u * LANES * DIM + d)], vals)

        def start_in(blk, buf):
            return pltpu.async_copy(
                wx_t.at[:, pl.ds(pl.multiple_of(blk * VB, VB), VB)],
                inb[buf], isem[buf])

        pltpu.sync_copy(
            center_hbm.at[pl.ds(pl.multiple_of(wid * b_per_w, 8), b_per_w)],
            cidx_v)
        for c in range(b_per_w // LANES):
            cvals = cidx_v[pl.ds(c * LANES, LANES)]
            descs, voffs = [], []
            for i in range(LANES):
                v = jnp.sum(jnp.where(lane == i, cvals, 0))
                vt = pl.multiple_of((v // VB) * VB, VB)
                voffs.append(v % VB)
                descs.append(pltpu.async_copy(
                    wc_t.at[:, pl.ds(vt, VB)],
                    cblk_v.at[:, pl.ds(i * VB, VB)], csem))
            for i, dsc in enumerate(descs):
                dsc.wait()
                crow = plsc.load_gather(
                    cblk_v, [lane, jnp.full((LANES,), i * VB, jnp.int32)
                             + voffs[i]])
                plsc.store_scatter(
                    crows_v, [(c * LANES + i) * DIM + lane], crow)
        pltpu.sync_copy(
            crows_v,
            crows_lin.at[pl.ds(
                pl.multiple_of(wid * b_per_w * DIM, 8), b_per_w * DIM)])

        start_in(lo, 0)
        start_in(lo + 1, 1)

        def step(j, carry):
            for sub in range(2):
                blk = lo + 2 * j + sub
                pltpu.make_async_copy(
                    wx_t.at[:, pl.ds(0, VB)], inb[sub], isem[sub]).wait()

                @pl.when(j > 0)
                def _drain_out(sub=sub):
                    pltpu.make_async_copy(
                        outb[sub], wx_lin.at[pl.ds(0, VB * DIM)],
                        osem[sub]).wait()

                transpose_block(inb[sub], outb[sub], VB)

                @pl.when(2 * j + sub + 2 < per_w)
                def _next_in(blk=blk, sub=sub):
                    pltpu.make_async_copy(
                        wx_t.at[:, pl.ds(
                            pl.multiple_of((blk + 2) * VB, VB), VB)],
                        inb[sub], isem[sub]).start()

                pltpu.make_async_copy(
                    outb[sub],
                    wx_lin.at[pl.ds(
                        pl.multiple_of(blk * VB * DIM, VB * DIM), VB * DIM)],
                    osem[sub]).start()
            return carry

        lax.fori_loop(0, per_w // 2, step, 0)
        pltpu.make_async_copy(ob0, wx_lin.at[pl.ds(0, VB * DIM)], os0).wait()
        pltpu.make_async_copy(ob1, wx_lin.at[pl.ds(0, VB * DIM)], os1).wait()

        for w in range(extra):
            @pl.when(wid == w)
            def _extra(w=w):
                blk = n_full - extra + w
                pltpu.sync_copy(wx_t.at[:, pl.ds(blk * VB, VB)], in0)
                transpose_block(in0, ob0, VB)
                pltpu.sync_copy(
                    ob0, wx_lin.at[pl.ds(blk * VB * DIM, VB * DIM)])

        if tail:
            @pl.when(wid == extra)
            def _tail():
                pltpu.sync_copy(wtail_hbm, tailb)
                for r in range(tail):
                    plsc.store_scatter(ob0, [r * DIM + lane], tailb[r, :])
                pltpu.sync_copy(
                    ob0.at[pl.ds(0, tail * DIM)],
                    wx_lin.at[pl.ds(n_full * VB * DIM, tail * DIM)])

    return relayout


def _make_main(B, V):
    b_per_w = B // NUM_WORKERS
    n_groups = b_per_w // LANES
    rows_per_group = LANES * CTX
    chunks = [(o, min(128, rows_per_group - o))
              for o in range(0, rows_per_group, 128)]

    mesh = plsc.VectorSubcoreMesh(core_axis_name="c", subcore_axis_name="s")

    @functools.partial(
        pl.kernel,
        out_type=jax.ShapeDtypeStruct((B * CTX,), jnp.float32),
        mesh=mesh,
        compiler_params=pltpu.CompilerParams(
            needs_layout_passes=False, use_tc_tiling_on_sc=False),
        scratch_types=[
            pltpu.VMEM((b_per_w * DIM,), jnp.float32),
            pltpu.VMEM((b_per_w * CTX,), jnp.int32),
            pltpu.VMEM((rows_per_group, DIM), jnp.float32),
            pltpu.VMEM((rows_per_group, DIM), jnp.float32),
            pltpu.VMEM((rows_per_group,), jnp.float32),
            pltpu.VMEM((rows_per_group,), jnp.float32),
            pltpu.SemaphoreType.DMA,
            pltpu.SemaphoreType.DMA,
            pltpu.SemaphoreType.DMA,
            pltpu.SemaphoreType.DMA,
        ],
    )
    def word_embed(context_hbm, crows_hbm, wx_hbm, out_hbm,
                   crows_v, xidx_v, xrows0, xrows1,
                   outv0, outv1, gs0, gs1, os0, os1):
        wid = lax.axis_index("s") * NUM_CORES + lax.axis_index("c")
        base = wid * b_per_w
        xrows = (xrows0, xrows1)
        outvs = (outv0, outv1)
        gsems = (gs0, gs1)
        osems = (os0, os1)
        lane = lax.iota(jnp.int32, LANES)

        pltpu.sync_copy(context_hbm.at[pl.ds(base * CTX, b_per_w * CTX)],
                        xidx_v)
        pltpu.sync_copy(crows_hbm.at[pl.ds(base * DIM, b_per_w * DIM)],
                        crows_v)

        def fire_group(g, buf):
            descs = []
            for off, n in chunks:
                idx = xidx_v.at[pl.ds(g * rows_per_group + off, n)]
                descs.append(pltpu.async_copy(
                    wx_hbm.at[idx], xrows[buf].at[pl.ds(off, n)],
                    gsems[buf]))
            return descs

        def compute_group(g, buf):
            rows = xrows[buf]
            ov = outvs[buf]
            ccols = [plsc.load_gather(
                crows_v, [(g * LANES + lane) * DIM + d]) for d in range(DIM)]

            def body(m, _):
                row_idx = lane * CTX + m
                acc = jnp.zeros((LANES,), jnp.float32)
                for d in range(DIM):
                    xcol = plsc.load_gather(
                        rows, [row_idx, jnp.full((LANES,), d, jnp.int32)])
                    acc = acc + ccols[d] * xcol
                plsc.store_scatter(ov, [row_idx], acc)
                return _

            lax.fori_loop(0, CTX, body, 0, unroll=2)

        inflight = {}
        out_descs = {}
        inflight[0] = fire_group(0, 0)
        for g in range(n_groups):
            buf = g % 2
            if g + 1 < n_groups:
                inflight[g + 1] = fire_group(g + 1, (g + 1) % 2)
            for dsc in inflight.pop(g):
                dsc.wait()
            if g - 2 in out_descs:
                out_descs.pop(g - 2).wait()
            compute_group(g, buf)
            out_descs[g] = pltpu.async_copy(
                outvs[buf],
                out_hbm.at[pl.ds((base + g * LANES) * CTX, rows_per_group)],
                osems[buf])
        for dsc in out_descs.values():
            dsc.wait()

    return word_embed


def kernel(center, context, W_center, W_context):
    B = center.shape[0]
    V = W_center.shape[0]
    wx_lin, crows_lin = _make_relayout(B, V)(
        W_context.T, W_center.T, center.reshape(B).astype(jnp.int32),
        W_context[V - (V % VB if V % VB else VB):, :])
    out_flat = _make_main(B, V)(
        context.reshape(B * CTX).astype(jnp.int32),
        crows_lin,
        wx_lin.reshape(V, DIM))
    return out_flat.reshape(B, 1, CTX)


# speedup vs baseline: 7.8942x; 1.2063x over previous
import functools

import jax
import jax.numpy as jnp
from jax import lax
from jax.experimental import pallas as pl
from jax.experimental.pallas import tpu as pltpu
from jax.experimental.pallas import tpu_sc as plsc

DIM = 16
CTX = 50
LANES = 16
NUM_CORES = 2
NUM_SUBCORES = 16
NUM_WORKERS = NUM_CORES * NUM_SUBCORES

VB = 512
TILEW = 128


def _make_relayout(B, V):
    n_full = V // VB
    per2 = 2 * (n_full // NUM_WORKERS // 2)
    extras = list(range(NUM_WORKERS * per2, n_full))
    tail = V % VB
    b_per_w = B // NUM_WORKERS

    mesh = plsc.VectorSubcoreMesh(core_axis_name="c", subcore_axis_name="s")

    @functools.partial(
        pl.kernel,
        out_type=(jax.ShapeDtypeStruct((V * DIM,), jnp.float32),
                  jax.ShapeDtypeStruct((B * DIM,), jnp.float32)),
        name="relayout",
        mesh=mesh,
        compiler_params=pltpu.CompilerParams(
            needs_layout_passes=False, use_tc_tiling_on_sc=True),
        scratch_types=[
            pltpu.VMEM((DIM, VB), jnp.float32),
            pltpu.VMEM((DIM, VB), jnp.float32),
            pltpu.VMEM((VB * DIM,), jnp.float32),
            pltpu.VMEM((VB * DIM,), jnp.float32),
            pltpu.VMEM((b_per_w,), jnp.int32),
            pltpu.VMEM((DIM, 16 * TILEW), jnp.float32),
            pltpu.VMEM((b_per_w * DIM,), jnp.float32),
            pltpu.VMEM((V % VB if V % VB else 8, DIM), jnp.float32),
            pltpu.SemaphoreType.DMA,
            pltpu.SemaphoreType.DMA,
            pltpu.SemaphoreType.DMA,
            pltpu.SemaphoreType.DMA,
            pltpu.SemaphoreType.DMA,
        ],
    )
    def relayout(wx_t, wc_t, center_hbm, wtail_hbm, wx_lin, crows_lin,
                 in0, in1, ob0, ob1, cidx_v, cblk_v, crows_v, tailb,
                 is0, is1, os0, os1, csem):
        wid = lax.axis_index("s") * NUM_CORES + lax.axis_index("c")
        lane = lax.iota(jnp.int32, LANES)
        inb = (in0, in1)
        outb = (ob0, ob1)
        isem = (is0, is1)
        osem = (os0, os1)
        lo = wid * per2
        idx16 = [lane * DIM + d for d in range(DIM)]

        def transpose_block(src, dst, nv):
            def tb(u, carry):
                dslice = dst.at[pl.ds(
                    pl.multiple_of(u * LANES * DIM, LANES * DIM),
                    LANES * DIM)]
                uoff = pl.multiple_of(u * LANES, LANES)
                for d in range(DIM):
                    vals = src[d, pl.ds(uoff, LANES)]
                    plsc.store_scatter(dslice, [idx16[d]], vals)
                return carry
            lax.fori_loop(0, nv // LANES, tb, 0, unroll=2)

        def start_in(blk, buf):
            return pltpu.async_copy(
                wx_t.at[:, pl.ds(pl.multiple_of(blk * VB, VB), VB)],
                inb[buf], isem[buf])

        pltpu.sync_copy(
            center_hbm.at[pl.ds(pl.multiple_of(wid * b_per_w, 8), b_per_w)],
            cidx_v)
        for c in range(b_per_w // LANES):
            cvals = cidx_v[pl.ds(c * LANES, LANES)]
            descs, voffs = [], []
            for i in range(LANES):
                v = jnp.sum(jnp.where(lane == i, cvals, 0))
                vt = pl.multiple_of((v // TILEW) * TILEW, TILEW)
                voffs.append(v % TILEW)
                descs.append(pltpu.async_copy(
                    wc_t.at[:, pl.ds(vt, TILEW)],
                    cblk_v.at[:, pl.ds(i * TILEW, TILEW)], csem))
            for i, dsc in enumerate(descs):
                dsc.wait()
                crow = plsc.load_gather(
                    cblk_v, [lane, jnp.full((LANES,), i * TILEW, jnp.int32)
                             + voffs[i]])
                plsc.store_scatter(
                    crows_v, [(c * LANES + i) * DIM + lane], crow)
        pltpu.sync_copy(
            crows_v,
            crows_lin.at[pl.ds(
                pl.multiple_of(wid * b_per_w * DIM, 8), b_per_w * DIM)])

        start_in(lo, 0)
        start_in(lo + 1, 1)

        def step(j, carry):
            for sub in range(2):
                blk = lo + 2 * j + sub
                pltpu.make_async_copy(
                    wx_t.at[:, pl.ds(0, VB)], inb[sub], isem[sub]).wait()

                @pl.when(j > 0)
                def _drain_out(sub=sub):
                    pltpu.make_async_copy(
                        outb[sub], wx_lin.at[pl.ds(0, VB * DIM)],
                        osem[sub]).wait()

                transpose_block(inb[sub], outb[sub], VB)

                @pl.when(2 * j + sub + 2 < per2)
                def _next_in(blk=blk, sub=sub):
                    pltpu.make_async_copy(
                        wx_t.at[:, pl.ds(
                            pl.multiple_of((blk + 2) * VB, VB), VB)],
                        inb[sub], isem[sub]).start()

                pltpu.make_async_copy(
                    outb[sub],
                    wx_lin.at[pl.ds(
                        pl.multiple_of(blk * VB * DIM, VB * DIM), VB * DIM)],
                    osem[sub]).start()
            return carry

        lax.fori_loop(0, per2 // 2, step, 0)
        pltpu.make_async_copy(ob0, wx_lin.at[pl.ds(0, VB * DIM)], os0).wait()
        pltpu.make_async_copy(ob1, wx_lin.at[pl.ds(0, VB * DIM)], os1).wait()

        n_extra = len(extras)
        assert n_extra <= 2 * NUM_WORKERS
        def do_extra(blk):
            boff = pl.multiple_of(blk * VB, VB)
            pltpu.sync_copy(wx_t.at[:, pl.ds(boff, VB)], in0)
            transpose_block(in0, ob0, VB)
            pltpu.sync_copy(
                ob0, wx_lin.at[pl.ds(
                    pl.multiple_of(blk * VB * DIM, VB * DIM), VB * DIM)])

        if n_extra:
            @pl.when(wid < min(n_extra, NUM_WORKERS))
            def _extra1():
                do_extra(extras[0] + wid)
        if n_extra > NUM_WORKERS:
            @pl.when(wid < n_extra - NUM_WORKERS)
            def _extra2():
                do_extra(extras[0] + NUM_WORKERS + wid)

        if tail:
            @pl.when(wid == 1)
            def _tail():
                pltpu.sync_copy(wtail_hbm, tailb)
                for r in range(tail):
                    plsc.store_scatter(ob0, [r * DIM + lane], tailb[r, :])
                pltpu.sync_copy(
                    ob0.at[pl.ds(0, tail * DIM)],
                    wx_lin.at[pl.ds(n_full * VB * DIM, tail * DIM)])

    return relayout


def _make_main(B, V):
    b_per_w = B // NUM_WORKERS
    n_groups = b_per_w // LANES
    rows_per_group = LANES * CTX
    chunks = [(o, min(128, rows_per_group - o))
              for o in range(0, rows_per_group, 128)]

    mesh = plsc.VectorSubcoreMesh(core_axis_name="c", subcore_axis_name="s")

    @functools.partial(
        pl.kernel,
        out_type=jax.ShapeDtypeStruct((B * CTX,), jnp.float32),
        mesh=mesh,
        compiler_params=pltpu.CompilerParams(
            needs_layout_passes=False, use_tc_tiling_on_sc=False),
        scratch_types=[
            pltpu.VMEM((b_per_w * DIM,), jnp.float32),
            pltpu.VMEM((b_per_w * CTX,), jnp.int32),
            pltpu.VMEM((rows_per_group, DIM), jnp.float32),
            pltpu.VMEM((rows_per_group, DIM), jnp.float32),
            pltpu.VMEM((rows_per_group,), jnp.float32),
            pltpu.VMEM((rows_per_group,), jnp.float32),
            pltpu.SemaphoreType.DMA,
            pltpu.SemaphoreType.DMA,
            pltpu.SemaphoreType.DMA,
            pltpu.SemaphoreType.DMA,
        ],
    )
    def word_embed(context_hbm, crows_hbm, wx_hbm, out_hbm,
                   crows_v, xidx_v, xrows0, xrows1,
                   outv0, outv1, gs0, gs1, os0, os1):
        wid = lax.axis_index("s") * NUM_CORES + lax.axis_index("c")
        base = wid * b_per_w
        xrows = (xrows0, xrows1)
        outvs = (outv0, outv1)
        gsems = (gs0, gs1)
        osems = (os0, os1)
        lane = lax.iota(jnp.int32, LANES)

        pltpu.sync_copy(context_hbm.at[pl.ds(base * CTX, b_per_w * CTX)],
                        xidx_v)
        pltpu.sync_copy(crows_hbm.at[pl.ds(base * DIM, b_per_w * DIM)],
                        crows_v)

        def fire_group(g, buf):
            descs = []
            for off, n in chunks:
                idx = xidx_v.at[pl.ds(g * rows_per_group + off, n)]
                descs.append(pltpu.async_copy(
                    wx_hbm.at[idx], xrows[buf].at[pl.ds(off, n)],
                    gsems[buf]))
            return descs

        def compute_group(g, buf):
            rows = xrows[buf]
            ov = outvs[buf]
            ccols = [plsc.load_gather(
                crows_v, [(g * LANES + lane) * DIM + d]) for d in range(DIM)]

            def body(m, _):
                row_idx = lane * CTX + m
                acc = jnp.zeros((LANES,), jnp.float32)
                for d in range(DIM):
                    xcol = plsc.load_gather(
                        rows, [row_idx, jnp.full((LANES,), d, jnp.int32)])
                    acc = acc + ccols[d] * xcol
                plsc.store_scatter(ov, [row_idx], acc)
                return _

            lax.fori_loop(0, CTX, body, 0, unroll=2)

        inflight = {}
        out_descs = {}
        inflight[0] = fire_group(0, 0)
        for g in range(n_groups):
            buf = g % 2
            if g + 1 < n_groups:
                inflight[g + 1] = fire_group(g + 1, (g + 1) % 2)
            for dsc in inflight.pop(g):
                dsc.wait()
            if g - 2 in out_descs:
                out_descs.pop(g - 2).wait()
            compute_group(g, buf)
            out_descs[g] = pltpu.async_copy(
                outvs[buf],
                out_hbm.at[pl.ds((base + g * LANES) * CTX, rows_per_group)],
                osems[buf])
        for dsc in out_descs.values():
            dsc.wait()

    return word_embed


def kernel(center, context, W_center, W_context):
    B = center.shape[0]
    V = W_center.shape[0]
    wx_lin, crows_lin = _make_relayout(B, V)(
        W_context.T, W_center.T, center.reshape(B).astype(jnp.int32),
        W_context[V - (V % VB if V % VB else VB):, :])
    out_flat = _make_main(B, V)(
        context.reshape(B * CTX).astype(jnp.int32),
        crows_lin,
        wx_lin.reshape(V, DIM))
    return out_flat.reshape(B, 1, CTX)
